# baseline (speedup 1.0000x reference)
import functools

import jax
import jax.numpy as jnp
from jax import lax
from jax.experimental import pallas as pl
from jax.experimental.pallas import tpu as pltpu
from jax.experimental.pallas import tpu_sc as plsc

T = 16384
D = 2048
NE = 64
K = 8
NW = 32
L = 16
BT = 1024
NCHUNKS = 2

_sc_mesh = plsc.VectorSubcoreMesh(core_axis_name="c", subcore_axis_name="s")


def _make_tc_chunk(c, nchunks):
    tc_tokens = T // nchunks
    sw = tc_tokens // NW
    spb = BT // sw
    blocks = tc_tokens // BT

    def body(x_ref, w_ref, o_ref):
        for s in range(spb):
            o_ref[s] = lax.dot_general(
                w_ref[...], x_ref[pl.ds(s * sw, sw), :],
                dimension_numbers=(((1,), (1,)), ((), ())),
                preferred_element_type=jnp.float32,
            )

    return pl.pallas_call(
        body,
        grid=(blocks,),
        in_specs=[
            pl.BlockSpec((BT, D), lambda i, _c=c, _b=blocks: (i + _c * _b, 0)),
            pl.BlockSpec((NE, D), lambda i: (0, 0)),
        ],
        out_specs=pl.BlockSpec((spb, NE, sw), lambda i: (i, 0, 0)),
        out_shape=jax.ShapeDtypeStruct((NW, NE, sw), jnp.float32),
    )


def _make_sc_chunk(c, nchunks):
    sw = T // nchunks // NW
    groups = sw // L
    chunk_el0 = c * (T // nchunks) * K

    @functools.partial(
        pl.kernel,
        mesh=_sc_mesh,
        out_type=[],
        scratch_types=[
            pltpu.VMEM((NE, sw), jnp.float32),
            pltpu.VMEM((K * sw,), jnp.int32),
            pltpu.VMEM((K * sw,), jnp.float32),
            pltpu.VMEM((sw * K,), jnp.int32),
            pltpu.VMEM((sw * K,), jnp.float32),
        ],
        compiler_params=pltpu.CompilerParams(needs_layout_passes=False),
    )
    def sc_topk(logits_hbm, idx_hbm, w_hbm, slab, stg_i, stg_w, idx_v, w_v):
        wid = lax.axis_index("s") * 2 + lax.axis_index("c")
        pltpu.sync_copy(logits_hbm.at[wid], slab)

        def group(g, carry):
            base = g * L
            tops = [jnp.full((L,), -jnp.inf, jnp.float32) for _ in range(K)]
            tids = [jnp.zeros((L,), jnp.int32) for _ in range(K)]
            for e in range(NE):
                v = slab[e, pl.ds(base, L)]
                vid = jnp.full((L,), e, jnp.int32)
                for i in range(K):
                    m = v > tops[i]
                    tv, ti = tops[i], tids[i]
                    tops[i] = jnp.where(m, v, tv)
                    tids[i] = jnp.where(m, vid, ti)
                    v = jnp.where(m, tv, v)
                    vid = jnp.where(m, ti, vid)
            mx = tops[0]
            es = [jnp.exp(t - mx) for t in tops]
            s = es[0]
            for i in range(1, K):
                s = s + es[i]
            inv = 1.0 / s
            for i in range(K):
                stg_i[pl.ds(i * sw + base, L)] = tids[i]
                stg_w[pl.ds(i * sw + base, L)] = es[i] * inv
            lane = lax.broadcasted_iota(jnp.int32, (L,), 0)
            for j in range(K):
                p = j * L + lane
                src = (p & (K - 1)) * sw + base + (p >> 3)
                dst = base * K + j * L
                idx_v[pl.ds(dst, L)] = plsc.load_gather(stg_i, [src])
                w_v[pl.ds(dst, L)] = plsc.load_gather(stg_w, [src])
            return carry

        lax.fori_loop(0, groups, group, 0)
        el0 = chunk_el0 + wid * (sw * K)
        pltpu.sync_copy(idx_v, idx_hbm.at[pl.ds(el0, sw * K)])
        pltpu.sync_copy(w_v, w_hbm.at[pl.ds(el0, sw * K)])

    return sc_topk


_tc_chunks = [_make_tc_chunk(c, NCHUNKS) for c in range(NCHUNKS)]
_sc_chunks = [_make_sc_chunk(c, NCHUNKS) for c in range(NCHUNKS)]


def kernel(x, top_k, W):
    del top_k
    idx_buf = jax.new_ref(jnp.zeros((T * K,), jnp.int32))
    w_buf = jax.new_ref(jnp.zeros((T * K,), jnp.float32))
    for c in range(NCHUNKS):
        logits = _tc_chunks[c](x, W)
        _sc_chunks[c](logits, idx_buf, w_buf)
    return idx_buf[...].reshape(T, K), w_buf[...].reshape(T, K)


# speedup vs baseline: 1.0328x; 1.0328x over previous
import functools

import jax
import jax.numpy as jnp
from jax import lax
from jax.experimental import pallas as pl
from jax.experimental.pallas import tpu as pltpu
from jax.experimental.pallas import tpu_sc as plsc

T = 16384
D = 2048
NE = 64
K = 8
NW = 32
L = 16
BT = 1024
NCHUNKS = 2
CT = T // NCHUNKS
SW = CT // NW
BPC = CT // BT

_sc_mesh = plsc.VectorSubcoreMesh(core_axis_name="c", subcore_axis_name="s")


def _make_tc_chunk(c):
    spb = BT // SW

    def body(x_ref, w_ref, o_ref):
        for s in range(spb):
            o_ref[s] = lax.dot_general(
                w_ref[...], x_ref[pl.ds(s * SW, SW), :],
                dimension_numbers=(((1,), (1,)), ((), ())),
                preferred_element_type=jnp.float32,
            )

    return pl.pallas_call(
        body,
        grid=(BPC,),
        in_specs=[
            pl.BlockSpec((BT, D), lambda i, _c=c: (i + _c * BPC, 0)),
            pl.BlockSpec((NE, D), lambda i: (0, 0)),
        ],
        out_specs=pl.BlockSpec((spb, NE, SW), lambda i: (i, 0, 0)),
        out_shape=jax.ShapeDtypeStruct((NW, NE, SW), jnp.float32),
    )


@functools.partial(
    pl.kernel,
    mesh=_sc_mesh,
    out_type=[
        jax.ShapeDtypeStruct((K, CT), jnp.int32),
        jax.ShapeDtypeStruct((K, CT), jnp.float32),
    ],
    scratch_types=[
        pltpu.VMEM((NE, SW), jnp.float32),
        pltpu.VMEM((K, SW), jnp.int32),
        pltpu.VMEM((K, SW), jnp.float32),
    ],
    compiler_params=pltpu.CompilerParams(needs_layout_passes=False),
)
def _sc_topk(logits_hbm, idx_hbm, w_hbm, slab, stg_i, stg_w):
    wid = lax.axis_index("s") * 2 + lax.axis_index("c")
    pltpu.sync_copy(logits_hbm.at[wid], slab)

    def group(g, carry):
        base = g * L
        tops = [jnp.full((L,), -jnp.inf, jnp.float32) for _ in range(K)]
        tids = [jnp.zeros((L,), jnp.int32) for _ in range(K)]
        for e in range(NE):
            v = slab[e, pl.ds(base, L)]
            vid = jnp.full((L,), e, jnp.int32)
            for i in range(K):
                m = v > tops[i]
                tv, ti = tops[i], tids[i]
                tops[i] = jnp.where(m, v, tv)
                tids[i] = jnp.where(m, vid, ti)
                v = jnp.where(m, tv, v)
                vid = jnp.where(m, ti, vid)
        mx = tops[0]
        es = [jnp.exp(t - mx) for t in tops]
        s = es[0]
        for i in range(1, K):
            s = s + es[i]
        inv = 1.0 / s
        for i in range(K):
            stg_i[i, pl.ds(base, L)] = tids[i]
            stg_w[i, pl.ds(base, L)] = es[i] * inv
        return carry

    lax.fori_loop(0, SW // L, group, 0)
    col0 = wid * SW
    pltpu.sync_copy(stg_i, idx_hbm.at[:, pl.ds(col0, SW)])
    pltpu.sync_copy(stg_w, w_hbm.at[:, pl.ds(col0, SW)])


def _packer_body(i1_ref, i2_ref, w1_ref, w2_ref, oi_ref, ow_ref):
    c = pl.program_id(0)
    first = c < BPC
    ii = jnp.where(first, i1_ref[...], i2_ref[...])
    ww = jnp.where(first, w1_ref[...], w2_ref[...])
    oi_ref[...] = ii.T
    ow_ref[...] = ww.T


_packer = pl.pallas_call(
    _packer_body,
    grid=(2 * BPC,),
    in_specs=[
        pl.BlockSpec((K, BT), lambda i: (0, jnp.minimum(i, BPC - 1))),
        pl.BlockSpec((K, BT), lambda i: (0, jnp.maximum(i - BPC, 0))),
        pl.BlockSpec((K, BT), lambda i: (0, jnp.minimum(i, BPC - 1))),
        pl.BlockSpec((K, BT), lambda i: (0, jnp.maximum(i - BPC, 0))),
    ],
    out_specs=[
        pl.BlockSpec((BT, K), lambda i: (i, 0)),
        pl.BlockSpec((BT, K), lambda i: (i, 0)),
    ],
    out_shape=[
        jax.ShapeDtypeStruct((T, K), jnp.int32),
        jax.ShapeDtypeStruct((T, K), jnp.float32),
    ],
)

_tc_chunks = [_make_tc_chunk(c) for c in range(NCHUNKS)]


def kernel(x, top_k, W):
    del top_k
    idx_c, w_c = [], []
    for c in range(NCHUNKS):
        logits = _tc_chunks[c](x, W)
        i_c, ww_c = _sc_topk(logits)
        idx_c.append(i_c)
        w_c.append(ww_c)
    return _packer(idx_c[0], idx_c[1], w_c[0], w_c[1])


# speedup vs baseline: 1.3200x; 1.2780x over previous
import functools

import jax
import jax.numpy as jnp
from jax import lax
from jax.experimental import pallas as pl
from jax.experimental.pallas import tpu as pltpu
from jax.experimental.pallas import tpu_sc as plsc

T = 16384
D = 2048
NE = 64
K = 8
NW = 32
L = 16
BT = 1024
NCHUNKS = 2
CT = T // NCHUNKS
SW = CT // NW
BPC = CT // BT

_sc_mesh = plsc.VectorSubcoreMesh(core_axis_name="c", subcore_axis_name="s")


def _make_tc_chunk(c):
    spb = BT // SW

    def body(x_ref, w_ref, o_ref):
        for s in range(spb):
            o_ref[s] = lax.dot_general(
                w_ref[...], x_ref[pl.ds(s * SW, SW), :],
                dimension_numbers=(((1,), (1,)), ((), ())),
                preferred_element_type=jnp.float32,
            )

    return pl.pallas_call(
        body,
        grid=(BPC,),
        in_specs=[
            pl.BlockSpec((BT, D), lambda i, _c=c: (i + _c * BPC, 0)),
            pl.BlockSpec((NE, D), lambda i: (0, 0)),
        ],
        out_specs=pl.BlockSpec((spb, NE, SW), lambda i: (i, 0, 0)),
        out_shape=jax.ShapeDtypeStruct((NW, NE, SW), jnp.float32),
    )


@functools.partial(
    pl.kernel,
    mesh=_sc_mesh,
    out_type=[
        jax.ShapeDtypeStruct((K, CT), jnp.int32),
        jax.ShapeDtypeStruct((K, CT), jnp.float32),
    ],
    scratch_types=[
        pltpu.VMEM((NE, SW), jnp.float32),
        pltpu.VMEM((K, SW), jnp.int32),
        pltpu.VMEM((K, SW), jnp.float32),
    ],
    compiler_params=pltpu.CompilerParams(needs_layout_passes=False),
)
def _sc_topk(logits_hbm, idx_hbm, w_hbm, slab, stg_i, stg_w):
    wid = lax.axis_index("s") * 2 + lax.axis_index("c")
    pltpu.sync_copy(logits_hbm.at[wid], slab)

    def group(g, carry):
        base = g * L
        tops = [jnp.full((L,), -jnp.inf, jnp.float32) for _ in range(K)]
        tids = [jnp.zeros((L,), jnp.int32) for _ in range(K)]
        for e in range(NE):
            v = slab[e, pl.ds(base, L)]
            vid = jnp.full((L,), e, jnp.int32)
            for i in range(K):
                m = v > tops[i]
                tv, ti = tops[i], tids[i]
                tops[i] = jnp.where(m, v, tv)
                tids[i] = jnp.where(m, vid, ti)
                v = jnp.where(m, tv, v)
                vid = jnp.where(m, ti, vid)
        mx = tops[0]
        es = [jnp.exp(t - mx) for t in tops]
        s = es[0]
        for i in range(1, K):
            s = s + es[i]
        inv = 1.0 / s
        for i in range(K):
            stg_i[i, pl.ds(base, L)] = tids[i]
            stg_w[i, pl.ds(base, L)] = es[i] * inv
        return carry

    lax.fori_loop(0, SW // L, group, 0)
    col0 = wid * SW
    pltpu.sync_copy(stg_i, idx_hbm.at[:, pl.ds(col0, SW)])
    pltpu.sync_copy(stg_w, w_hbm.at[:, pl.ds(col0, SW)])


_tc_chunks = [_make_tc_chunk(c) for c in range(NCHUNKS)]


def kernel(x, top_k, W):
    del top_k
    idx_c, w_c = [], []
    for c in range(NCHUNKS):
        logits = _tc_chunks[c](x, W)
        i_c, ww_c = _sc_topk(logits)
        idx_c.append(i_c)
        w_c.append(ww_c)
    idx = jnp.concatenate(idx_c, axis=1).T
    w = jnp.concatenate(w_c, axis=1).T
    return idx, w
